# baseline (speedup 1.0000x reference)
import functools

import jax
import jax.numpy as jnp
from jax import lax
from jax.experimental import pallas as pl
from jax.experimental.pallas import tpu as pltpu
from jax.experimental.pallas import tpu_sc as plsc

N_LIT = 10000
N_CL = 10000
N_EDGE = 320000
F = 128
FA = F + 16

NC = 2
NS = 16
NW = NC * NS
E_PER_W = N_EDGE // NW
CHUNK = 96
N_CHUNK = 105
E_PAD_W = N_CHUNK * CHUNK - E_PER_W
N_PAD = 10240
ROWS_PER_SUB = N_PAD // NS
DUMP_ROW = N_PAD - 8

ROW_BLK = 1000


def _lin1_body(x_ref, w_ref, b_ref, o_ref):
    mm = jnp.dot(x_ref[...], w_ref[...], preferred_element_type=jnp.float32)
    wh = mm + b_ref[...]
    ones = jnp.ones((wh.shape[0], FA - F), jnp.float32)
    o_ref[...] = jnp.concatenate([wh, ones], axis=1)


def _tc_lin1(x, w, b):
    n = x.shape[0]
    return pl.pallas_call(
        _lin1_body,
        grid=(n // ROW_BLK,),
        in_specs=[
            pl.BlockSpec((ROW_BLK, F), lambda i: (i, 0)),
            pl.BlockSpec((F, F), lambda i: (0, 0)),
            pl.BlockSpec((1, F), lambda i: (0, 0)),
        ],
        out_specs=pl.BlockSpec((ROW_BLK, FA), lambda i: (i, 0)),
        out_shape=jax.ShapeDtypeStruct((n, FA), jnp.float32),
    )(x, w, b.reshape(1, F))


def _mid_body(acc_ref, xc_ref, w_ref, b_ref, o_ref):
    a = acc_ref[0] + acc_ref[1]
    s = a[:, :F]
    deg = a[:, F:F + 1]
    cembs = jnp.maximum(s / jnp.maximum(deg, 1.0), 0.0)
    w_main = w_ref[:F, :]
    w_last = w_ref[F:F + 1, :]
    mm = jnp.dot(cembs, w_main, preferred_element_type=jnp.float32)
    wh = mm + xc_ref[...] * w_last + b_ref[...]
    ones = jnp.ones((wh.shape[0], FA - F), jnp.float32)
    o_ref[...] = jnp.concatenate([wh, ones], axis=1)


def _tc_mid(acc, x_clause, w, b):
    n = x_clause.shape[0]
    return pl.pallas_call(
        _mid_body,
        grid=(n // ROW_BLK,),
        in_specs=[
            pl.BlockSpec((2, ROW_BLK, FA), lambda i: (0, i, 0)),
            pl.BlockSpec((ROW_BLK, 1), lambda i: (i, 0)),
            pl.BlockSpec((F + 1, F), lambda i: (0, 0)),
            pl.BlockSpec((1, F), lambda i: (0, 0)),
        ],
        out_specs=pl.BlockSpec((ROW_BLK, FA), lambda i: (i, 0)),
        out_shape=jax.ShapeDtypeStruct((n, FA), jnp.float32),
    )(acc, x_clause, w, b.reshape(1, F))


def _final_body(acc_ref, o_ref):
    a = acc_ref[0] + acc_ref[1]
    s = a[:, :F]
    deg = a[:, F:F + 1]
    o_ref[...] = jnp.maximum(s / jnp.maximum(deg, 1.0), 0.0)


def _tc_final(acc):
    n = N_LIT
    return pl.pallas_call(
        _final_body,
        grid=(n // ROW_BLK,),
        in_specs=[pl.BlockSpec((2, ROW_BLK, FA), lambda i: (0, i, 0))],
        out_specs=pl.BlockSpec((ROW_BLK, F), lambda i: (i, 0)),
        out_shape=jax.ShapeDtypeStruct((n, F), jnp.float32),
    )(acc)


def _sc_aggregate(wh, g_idx, s_idx, zeros):
    mesh = plsc.VectorSubcoreMesh(core_axis_name="c", subcore_axis_name="s")

    @functools.partial(
        pl.kernel,
        mesh=mesh,
        out_type=jax.ShapeDtypeStruct((NC, N_PAD, FA), jnp.float32),
        scratch_types=[
            pltpu.VMEM((N_CHUNK, CHUNK), jnp.int32),
            pltpu.VMEM((N_CHUNK, CHUNK), jnp.int32),
            pltpu.VMEM((CHUNK, FA), jnp.float32),
            pltpu.VMEM_SHARED((N_PAD, FA), jnp.float32),
            pltpu.SemaphoreType.DMA,
            pltpu.SemaphoreType.DMA,
        ],
        compiler_params=pltpu.CompilerParams(use_tc_tiling_on_sc=False),
    )
    def k(wh_hbm, gi_hbm, si_hbm, z_hbm, out_hbm,
          gi_v, si_v, rows_v, acc_sh, sem0, sem1):
        cid = lax.axis_index("c")
        sid = lax.axis_index("s")
        wid = cid * NS + sid
        stripe = sid * ROWS_PER_SUB

        pltpu.sync_copy(z_hbm.at[pl.ds(stripe, ROWS_PER_SUB)],
                        acc_sh.at[pl.ds(stripe, ROWS_PER_SUB)])
        pltpu.sync_copy(gi_hbm.at[wid], gi_v)
        pltpu.sync_copy(si_hbm.at[wid], si_v)
        plsc.subcore_barrier()

        H = CHUNK // 2
        sems = (sem0, sem1)

        @pl.loop(0, N_CHUNK)
        def _(i):
            gs = [pltpu.async_copy(wh_hbm.at[gi_v.at[i, pl.ds(q * H, H)]],
                                   rows_v.at[pl.ds(q * H, H)], sems[q])
                  for q in range(2)]
            for c in gs:
                c.wait()
            ss = [pltpu.async_copy(rows_v.at[pl.ds(q * H, H)],
                                   acc_sh.at[si_v.at[i, pl.ds(q * H, H)]],
                                   sems[q], add=True)
                  for q in range(2)]
            for c in ss:
                c.wait()

        plsc.subcore_barrier()
        pltpu.sync_copy(acc_sh.at[pl.ds(stripe, ROWS_PER_SUB)],
                        out_hbm.at[cid, pl.ds(stripe, ROWS_PER_SUB)])

    return k(wh, g_idx, s_idx, zeros)


@jax.jit
def kernel(x_literal, x_clause, edge_index, W_l2c, b_l2c, W_c2l, b_c2l):
    src = edge_index[0].astype(jnp.int32).reshape(NW, E_PER_W)
    dst = edge_index[1].astype(jnp.int32).reshape(NW, E_PER_W)
    pad_g = jnp.zeros((NW, E_PAD_W), jnp.int32)
    pad_s = jnp.full((NW, E_PAD_W), DUMP_ROW, jnp.int32)

    def _padded(idx, pad):
        if E_PAD_W == 0:
            return idx.reshape(NW, N_CHUNK, CHUNK)
        return jnp.concatenate([idx, pad], axis=1).reshape(NW, N_CHUNK, CHUNK)

    src_g, src_s = _padded(src, pad_g), _padded(src, pad_s)
    dst_g, dst_s = _padded(dst, pad_g), _padded(dst, pad_s)
    zeros = jnp.zeros((N_PAD, FA), jnp.float32)

    wh1 = _tc_lin1(x_literal, W_l2c, b_l2c)
    acc1 = _sc_aggregate(wh1, src_g, dst_s, zeros)
    wh2 = _tc_mid(acc1, x_clause, W_c2l, b_c2l)
    acc2 = _sc_aggregate(wh2, dst_g, src_s, zeros)
    return _tc_final(acc2)


# speedup vs baseline: 1.3158x; 1.3158x over previous
import functools

import jax
import jax.numpy as jnp
from jax import lax
from jax.experimental import pallas as pl
from jax.experimental.pallas import tpu as pltpu
from jax.experimental.pallas import tpu_sc as plsc

N_LIT = 10000
N_CL = 10000
N_EDGE = 320000
F = 128
FA = F + 16

NC = 2
NS = 16
NW = NC * NS
E_PER_W = N_EDGE // NW
CHUNK = 80
N_CHUNK = 125
E_PAD_W = N_CHUNK * CHUNK - E_PER_W
N_PAD = 10240
ROWS_PER_SUB = N_PAD // NS
DUMP_ROW = N_PAD - 8

ROW_BLK = 1000


def _lin1_body(x_ref, w_ref, b_ref, o_ref):
    mm = jnp.dot(x_ref[...], w_ref[...], preferred_element_type=jnp.float32)
    wh = mm + b_ref[...]
    ones = jnp.ones((wh.shape[0], FA - F), jnp.float32)
    o_ref[...] = jnp.concatenate([wh, ones], axis=1)


def _tc_lin1(x, w, b):
    n = x.shape[0]
    return pl.pallas_call(
        _lin1_body,
        grid=(n // ROW_BLK,),
        in_specs=[
            pl.BlockSpec((ROW_BLK, F), lambda i: (i, 0)),
            pl.BlockSpec((F, F), lambda i: (0, 0)),
            pl.BlockSpec((1, F), lambda i: (0, 0)),
        ],
        out_specs=pl.BlockSpec((ROW_BLK, FA), lambda i: (i, 0)),
        out_shape=jax.ShapeDtypeStruct((n, FA), jnp.float32),
    )(x, w, b.reshape(1, F))


def _mid_body(acc_ref, xc_ref, w_ref, b_ref, o_ref):
    a = acc_ref[0] + acc_ref[1]
    s = a[:, :F]
    deg = a[:, F:F + 1]
    cembs = jnp.maximum(s / jnp.maximum(deg, 1.0), 0.0)
    w_main = w_ref[:F, :]
    w_last = w_ref[F:F + 1, :]
    mm = jnp.dot(cembs, w_main, preferred_element_type=jnp.float32)
    wh = mm + xc_ref[...] * w_last + b_ref[...]
    ones = jnp.ones((wh.shape[0], FA - F), jnp.float32)
    o_ref[...] = jnp.concatenate([wh, ones], axis=1)


def _tc_mid(acc, x_clause, w, b):
    n = x_clause.shape[0]
    return pl.pallas_call(
        _mid_body,
        grid=(n // ROW_BLK,),
        in_specs=[
            pl.BlockSpec((2, ROW_BLK, FA), lambda i: (0, i, 0)),
            pl.BlockSpec((ROW_BLK, 1), lambda i: (i, 0)),
            pl.BlockSpec((F + 1, F), lambda i: (0, 0)),
            pl.BlockSpec((1, F), lambda i: (0, 0)),
        ],
        out_specs=pl.BlockSpec((ROW_BLK, FA), lambda i: (i, 0)),
        out_shape=jax.ShapeDtypeStruct((n, FA), jnp.float32),
    )(acc, x_clause, w, b.reshape(1, F))


def _final_body(acc_ref, o_ref):
    a = acc_ref[0] + acc_ref[1]
    s = a[:, :F]
    deg = a[:, F:F + 1]
    o_ref[...] = jnp.maximum(s / jnp.maximum(deg, 1.0), 0.0)


def _tc_final(acc):
    n = N_LIT
    return pl.pallas_call(
        _final_body,
        grid=(n // ROW_BLK,),
        in_specs=[pl.BlockSpec((2, ROW_BLK, FA), lambda i: (0, i, 0))],
        out_specs=pl.BlockSpec((ROW_BLK, F), lambda i: (i, 0)),
        out_shape=jax.ShapeDtypeStruct((n, F), jnp.float32),
    )(acc)


def _sc_aggregate(wh, g_idx, s_idx, zeros):
    mesh = plsc.VectorSubcoreMesh(core_axis_name="c", subcore_axis_name="s")

    @functools.partial(
        pl.kernel,
        mesh=mesh,
        out_type=jax.ShapeDtypeStruct((NC, N_PAD, FA), jnp.float32),
        scratch_types=[
            pltpu.VMEM((N_CHUNK, CHUNK), jnp.int32),
            pltpu.VMEM((N_CHUNK, CHUNK), jnp.int32),
            pltpu.VMEM((CHUNK, FA), jnp.float32),
            pltpu.VMEM_SHARED((N_PAD, FA), jnp.float32),
            pltpu.SemaphoreType.DMA,
            pltpu.SemaphoreType.DMA,
        ],
        compiler_params=pltpu.CompilerParams(use_tc_tiling_on_sc=False),
    )
    def k(wh_hbm, gi_hbm, si_hbm, z_hbm, out_hbm,
          gi_v, si_v, rows_v, acc_sh, sem0, sem1):
        cid = lax.axis_index("c")
        sid = lax.axis_index("s")
        wid = cid * NS + sid
        stripe = sid * ROWS_PER_SUB

        pltpu.sync_copy(z_hbm.at[pl.ds(stripe, ROWS_PER_SUB)],
                        acc_sh.at[pl.ds(stripe, ROWS_PER_SUB)])
        pltpu.sync_copy(gi_hbm.at[wid], gi_v)
        pltpu.sync_copy(si_hbm.at[wid], si_v)
        plsc.subcore_barrier()

        H = CHUNK // 2
        sems = (sem0, sem1)

        @pl.loop(0, N_CHUNK)
        def _(i):
            gs = [pltpu.async_copy(wh_hbm.at[gi_v.at[i, pl.ds(q * H, H)]],
                                   rows_v.at[pl.ds(q * H, H)], sems[q])
                  for q in range(2)]
            for c in gs:
                c.wait()
            ss = [pltpu.async_copy(rows_v.at[pl.ds(q * H, H)],
                                   acc_sh.at[si_v.at[i, pl.ds(q * H, H)]],
                                   sems[q], add=True)
                  for q in range(2)]
            for c in ss:
                c.wait()

        plsc.subcore_barrier()
        pltpu.sync_copy(acc_sh.at[pl.ds(stripe, ROWS_PER_SUB)],
                        out_hbm.at[cid, pl.ds(stripe, ROWS_PER_SUB)])

    return k(wh, g_idx, s_idx, zeros)


@jax.jit
def kernel(x_literal, x_clause, edge_index, W_l2c, b_l2c, W_c2l, b_c2l):
    src = edge_index[0].astype(jnp.int32).reshape(NW, E_PER_W)
    dst = edge_index[1].astype(jnp.int32).reshape(NW, E_PER_W)
    pad_g = jnp.zeros((NW, E_PAD_W), jnp.int32)
    pad_s = jnp.full((NW, E_PAD_W), DUMP_ROW, jnp.int32)

    def _padded(idx, pad):
        if E_PAD_W == 0:
            return idx.reshape(NW, N_CHUNK, CHUNK)
        return jnp.concatenate([idx, pad], axis=1).reshape(NW, N_CHUNK, CHUNK)

    src_g, src_s = _padded(src, pad_g), _padded(src, pad_s)
    dst_g, dst_s = _padded(dst, pad_g), _padded(dst, pad_s)
    zeros = jnp.zeros((N_PAD, FA), jnp.float32)

    wh1 = _tc_lin1(x_literal, W_l2c, b_l2c)
    acc1 = _sc_aggregate(wh1, src_g, dst_s, zeros)
    wh2 = _tc_mid(acc1, x_clause, W_c2l, b_c2l)
    acc2 = _sc_aggregate(wh2, dst_g, src_s, zeros)
    return _tc_final(acc2)
